# baseline (speedup 1.0000x reference)
import functools

import jax
import jax.numpy as jnp
from jax import lax
from jax.experimental import pallas as pl
from jax.experimental.pallas import tpu as pltpu
from jax.experimental.pallas import tpu_sc as plsc

D = 64
SCALE = 8.0
NC, NS = 2, 16
NW = NC * NS
IDX_MINOR = 128
ROWS_PER_GATHER = 128
CHUNK = 512
LANES = 16


@functools.partial(jax.jit, static_argnums=(2, 3))
def _gather_scale(x_r, lut, n_total, n_per_w):
    n_idx_rows = n_per_w // IDX_MINOR
    g_per_chunk = CHUNK // ROWS_PER_GATHER
    n_chunks = n_per_w // CHUNK

    mesh = plsc.VectorSubcoreMesh(core_axis_name="c", subcore_axis_name="s")

    @functools.partial(
        pl.kernel,
        mesh=mesh,
        compiler_params=pltpu.CompilerParams(use_tc_tiling_on_sc=False),
        out_type=jax.ShapeDtypeStruct((n_total, D), jnp.float32),
        scratch_types=[
            pltpu.VMEM((n_idx_rows, IDX_MINOR), jnp.int32),
            pltpu.VMEM((CHUNK, D), jnp.float32),
            pltpu.SemaphoreType.DMA,
        ],
    )
    def k(x_hbm, lut_hbm, out_hbm, idx_v, rows_v, sem):
        wid = lax.axis_index("s") * NC + lax.axis_index("c")
        base = wid * n_per_w
        pltpu.sync_copy(x_hbm.at[wid], idx_v)

        def chunk_body(c, carry):
            copies = []
            for j in range(g_per_chunk):
                copies.append(
                    pltpu.async_copy(
                        lut_hbm.at[idx_v.at[c * g_per_chunk + j]],
                        rows_v.at[pl.ds(j * ROWS_PER_GATHER, ROWS_PER_GATHER)],
                        sem,
                    )
                )
            for cp in copies:
                cp.wait()

            def scale_body(i, acc):
                for j in range(D // LANES):
                    sl = pl.ds(j * LANES, LANES)
                    rows_v[i, sl] = rows_v[i, sl] * SCALE
                return acc

            lax.fori_loop(0, CHUNK, scale_body, 0)

            pltpu.sync_copy(
                rows_v, out_hbm.at[pl.ds(base + c * CHUNK, CHUNK)]
            )
            return carry

        lax.fori_loop(0, n_chunks, chunk_body, 0)

    return k(x_r, lut)


def kernel(x, lut):
    b, s = x.shape
    n_total = b * s
    n_per_w = n_total // NW
    x_r = x.reshape(NW, n_per_w // IDX_MINOR, IDX_MINOR)
    out = _gather_scale(x_r, lut, n_total, n_per_w)
    return out.reshape(b, s, D)


# speedup vs baseline: 1.1130x; 1.1130x over previous
import functools

import jax
import jax.numpy as jnp
from jax import lax
from jax.experimental import pallas as pl
from jax.experimental.pallas import tpu as pltpu
from jax.experimental.pallas import tpu_sc as plsc

D = 64
SCALE = 8.0
NC, NS = 2, 16
NW = NC * NS
IDX_MINOR = 128
ROWS_PER_GATHER = 128
CHUNK = 256
LANES = 16


@functools.partial(jax.jit, static_argnums=(2, 3))
def _gather_scale(x_r, lut, n_total, n_per_w):
    n_idx_rows = n_per_w // IDX_MINOR
    g_per_chunk = CHUNK // ROWS_PER_GATHER
    n_chunks = n_per_w // CHUNK
    assert n_chunks % 2 == 0 and n_chunks >= 6

    mesh = plsc.VectorSubcoreMesh(core_axis_name="c", subcore_axis_name="s")

    @functools.partial(
        pl.kernel,
        mesh=mesh,
        compiler_params=pltpu.CompilerParams(use_tc_tiling_on_sc=False),
        out_type=jax.ShapeDtypeStruct((n_total, D), jnp.float32),
        scratch_types=[
            pltpu.VMEM((n_idx_rows, IDX_MINOR), jnp.int32),
            pltpu.VMEM((CHUNK, D), jnp.float32),
            pltpu.VMEM((CHUNK, D), jnp.float32),
            pltpu.VMEM((CHUNK, D), jnp.float32),
            pltpu.VMEM((CHUNK, D), jnp.float32),
            pltpu.SemaphoreType.DMA,
            pltpu.SemaphoreType.DMA,
            pltpu.SemaphoreType.DMA,
            pltpu.SemaphoreType.DMA,
        ],
    )
    def k(x_hbm, lut_hbm, out_hbm, idx_v, ga, gb, sa, sb, gsa, gsb, ssa, ssb):
        wid = lax.axis_index("s") * NC + lax.axis_index("c")
        base = wid * n_per_w
        gbuf, sbuf = (ga, gb), (sa, sb)
        gsem, ssem = (gsa, gsb), (ssa, ssb)

        pltpu.sync_copy(x_hbm.at[wid], idx_v)

        def fire_gathers(c, b):
            for j in range(g_per_chunk):
                pltpu.async_copy(
                    lut_hbm.at[idx_v.at[c * g_per_chunk + j]],
                    gbuf[b].at[pl.ds(j * ROWS_PER_GATHER, ROWS_PER_GATHER)],
                    gsem[b],
                )

        def drain_gathers(b):
            pltpu.make_async_copy(
                lut_hbm.at[pl.ds(0, CHUNK)], gbuf[b], gsem[b]
            ).wait()

        def drain_store(b):
            pltpu.make_async_copy(
                lut_hbm.at[pl.ds(0, CHUNK)], sbuf[b], ssem[b]
            ).wait()

        def scale(b):
            src, dst = gbuf[b], sbuf[b]

            @plsc.parallel_loop(0, CHUNK, unroll=4)
            def _(i):
                for j in range(D // LANES):
                    sl = pl.ds(j * LANES, LANES)
                    dst[i, sl] = src[i, sl] * SCALE

        def fire_store(c, b):
            pltpu.async_copy(
                sbuf[b], out_hbm.at[pl.ds(base + c * CHUNK, CHUNK)], ssem[b]
            )

        fire_gathers(0, 0)
        fire_gathers(1, 1)
        for b in range(2):
            drain_gathers(b)
            scale(b)
            fire_store(b, b)
            fire_gathers(b + 2, b)

        def step(it, carry):
            c0 = 2 + 2 * it
            for b in range(2):
                c = c0 + b
                drain_gathers(b)
                drain_store(b)
                scale(b)
                fire_store(c, b)
                fire_gathers(c + 2, b)
            return carry

        lax.fori_loop(0, (n_chunks - 4) // 2, step, 0)

        for b in range(2):
            drain_gathers(b)
            drain_store(b)
            scale(b)
            fire_store(n_chunks - 2 + b, b)
        for b in range(2):
            drain_store(b)

    return k(x_r, lut)


def kernel(x, lut):
    b, s = x.shape
    n_total = b * s
    n_per_w = n_total // NW
    x_r = x.reshape(NW, n_per_w // IDX_MINOR, IDX_MINOR)
    out = _gather_scale(x_r, lut, n_total, n_per_w)
    return out.reshape(b, s, D)
